# baseline (speedup 1.0000x reference)
import functools

import jax
import jax.numpy as jnp
from jax import lax
from jax.experimental import pallas as pl
from jax.experimental.pallas import tpu as pltpu
from jax.experimental.pallas import tpu_sc as plsc

NC = 2
NS = 16
NBUF = 5
LEAD = 3
B = 80


def _scale_rows(rows_ref, wv, base):
    for g2 in range(B // 16):
        wvec = wv[pl.ds(base + g2 * 16, 16)]
        for i in range(16):
            e = g2 * 16 + i
            rows_ref[e, :] = rows_ref[e, :] * wvec[i]


def _fill_rows(rows_ref, wv, base):
    for g2 in range(B // 16):
        wvec = wv[pl.ds(base + g2 * 16, 16)]
        for i in range(16):
            rows_ref[g2 * 16 + i, :] = jnp.broadcast_to(wvec[i], (16,))


def _make_deg_kernel(N2, H, K, EW):
    rps = N2 // NS
    mesh = plsc.VectorSubcoreMesh(
        core_axis_name="c", subcore_axis_name="s", num_cores=NC, num_subcores=NS)

    @functools.partial(
        pl.kernel,
        out_type=jax.ShapeDtypeStruct((NC, N2, H), jnp.float32),
        mesh=mesh,
        scratch_types=(
            [pltpu.VMEM((EW,), jnp.int32),
             pltpu.VMEM((EW,), jnp.float32)]
            + [pltpu.VMEM((B, H), jnp.float32)] * NBUF
            + [pltpu.VMEM_SHARED((N2, H), jnp.float32)]
            + [pltpu.SemaphoreType.DMA] * NBUF
        ),
        compiler_params=pltpu.CompilerParams(use_tc_tiling_on_sc=False),
    )
    def deg_kernel(d_h, w_h, z_h, out_h, dv, wv, *rest):
        rows = rest[:NBUF]
        accn = rest[NBUF]
        ssem = rest[NBUF + 1:]
        c = lax.axis_index("c")
        sid = lax.axis_index("s")
        wid = sid * NC + c
        pltpu.sync_copy(z_h.at[pl.ds(sid * rps, rps)],
                        accn.at[pl.ds(sid * rps, rps)])
        plsc.subcore_barrier()
        pltpu.sync_copy(d_h.at[pl.ds(wid * EW, EW)], dv)
        pltpu.sync_copy(w_h.at[pl.ds(wid * EW, EW)], wv)

        def outer(g, carry):
            for b in range(NBUF):
                j = g * NBUF + b

                @pl.when(j >= NBUF)
                def _wait_prev():
                    pltpu.make_async_copy(
                        rows[b], accn.at[dv.at[pl.ds((j - NBUF) * B, B)]],
                        ssem[b]).wait()

                _fill_rows(rows[b], wv, j * B)
                pltpu.async_copy(rows[b], accn.at[dv.at[pl.ds(j * B, B)]],
                                 ssem[b], add=True)
            return carry

        lax.fori_loop(0, K // NBUF, outer, 0)
        for b in range(NBUF):
            pltpu.make_async_copy(
                rows[b], accn.at[dv.at[pl.ds((K - NBUF + b) * B, B)]],
                ssem[b]).wait()
        plsc.subcore_barrier()
        pltpu.sync_copy(accn.at[pl.ds(sid * rps, rps)],
                        out_h.at[c, pl.ds(sid * rps, rps)])

    return deg_kernel


def _make_msg_kernel(N, N2, H, K, EW):
    rps = N2 // NS
    mesh = plsc.VectorSubcoreMesh(
        core_axis_name="c", subcore_axis_name="s", num_cores=NC, num_subcores=NS)

    @functools.partial(
        pl.kernel,
        out_type=jax.ShapeDtypeStruct((NC, N2, H), jnp.float32),
        mesh=mesh,
        scratch_types=(
            [pltpu.VMEM((EW,), jnp.int32),
             pltpu.VMEM((EW,), jnp.int32),
             pltpu.VMEM((EW,), jnp.float32)]
            + [pltpu.VMEM((B, H), jnp.float32)] * NBUF
            + [pltpu.VMEM_SHARED((N2, H), jnp.float32)]
            + [pltpu.SemaphoreType.DMA] * NBUF
            + [pltpu.SemaphoreType.DMA] * NBUF
        ),
        compiler_params=pltpu.CompilerParams(use_tc_tiling_on_sc=False),
    )
    def msg_kernel(table_h, s_h, d_h, w_h, z_h, out_h, sv, dv, wv, *rest):
        rows = rest[:NBUF]
        acc = rest[NBUF]
        gsem = rest[NBUF + 1:NBUF + 1 + NBUF]
        ssem = rest[NBUF + 1 + NBUF:]
        c = lax.axis_index("c")
        sid = lax.axis_index("s")
        wid = sid * NC + c
        pltpu.sync_copy(z_h.at[pl.ds(sid * rps, rps)],
                        acc.at[pl.ds(sid * rps, rps)])
        plsc.subcore_barrier()
        pltpu.sync_copy(s_h.at[pl.ds(wid * EW, EW)], sv)
        pltpu.sync_copy(d_h.at[pl.ds(wid * EW, EW)], dv)
        pltpu.sync_copy(w_h.at[pl.ds(wid * EW, EW)], wv)

        for b in range(NBUF):
            pltpu.async_copy(table_h.at[sv.at[pl.ds(b * B, B)]],
                             rows[b], gsem[b])

        def outer(g, carry):
            for b in range(NBUF):
                j = g * NBUF + b
                bn = (b + LEAD) % NBUF

                @pl.when(jnp.logical_and(j + LEAD >= NBUF, j + LEAD < K))
                def _fire_next():
                    pltpu.make_async_copy(
                        rows[bn],
                        acc.at[dv.at[pl.ds((j + LEAD - NBUF) * B, B)]],
                        ssem[bn]).wait()
                    pltpu.async_copy(
                        table_h.at[sv.at[pl.ds((j + LEAD) * B, B)]],
                        rows[bn], gsem[bn])

                pltpu.make_async_copy(
                    table_h.at[sv.at[pl.ds(j * B, B)]], rows[b],
                    gsem[b]).wait()
                _scale_rows(rows[b], wv, j * B)
                pltpu.async_copy(rows[b], acc.at[dv.at[pl.ds(j * B, B)]],
                                 ssem[b], add=True)
            return carry

        lax.fori_loop(0, K // NBUF, outer, 0)
        for b in range(NBUF):
            pltpu.make_async_copy(
                rows[b], acc.at[dv.at[pl.ds((K - NBUF + b) * B, B)]],
                ssem[b]).wait()
        plsc.subcore_barrier()
        pltpu.sync_copy(acc.at[pl.ds(sid * rps, rps)],
                        out_h.at[c, pl.ds(sid * rps, rps)])

    return msg_kernel


def _tc1(degp, x, W1, N, N2, D, H, R):
    def body(degp_ref, x_ref, w1_ref, dis_ref, dinv_ref, h1_ref, t1_ref):
        deg = degp_ref[0, :, 0:1] + degp_ref[1, :, 0:1] + 1.0
        dis = lax.rsqrt(deg)
        dis_ref[...] = dis
        dinv_ref[...] = 1.0 / deg
        h1 = jnp.dot(x_ref[...], w1_ref[...], preferred_element_type=jnp.float32)
        h1_ref[...] = h1
        t1_ref[...] = dis * h1

    f32 = jnp.float32
    return pl.pallas_call(
        body,
        grid=(N // R,),
        in_specs=[
            pl.BlockSpec((NC, R, H), lambda i: (0, i, 0)),
            pl.BlockSpec((R, D), lambda i: (i, 0)),
            pl.BlockSpec((D, H), lambda i: (0, 0)),
        ],
        out_specs=(
            pl.BlockSpec((R, 1), lambda i: (i, 0)),
            pl.BlockSpec((R, 1), lambda i: (i, 0)),
            pl.BlockSpec((R, H), lambda i: (i, 0)),
            pl.BlockSpec((R, H), lambda i: (i, 0)),
        ),
        out_shape=(
            jax.ShapeDtypeStruct((N, 1), f32),
            jax.ShapeDtypeStruct((N, 1), f32),
            jax.ShapeDtypeStruct((N, H), f32),
            jax.ShapeDtypeStruct((N, H), f32),
        ),
    )(degp, x, W1)


def _tc2(acc1, h1, dis, dinv, b1, N, N2, H, R):
    def body(acc_ref, h1_ref, dis_ref, dinv_ref, b1_ref, z1_ref, t2_ref):
        agg = (dis_ref[...] * (acc_ref[0] + acc_ref[1])
               + dinv_ref[...] * h1_ref[...] + b1_ref[...])
        z1 = jnp.maximum(agg, 0.0)
        z1_ref[...] = z1
        t2_ref[...] = dis_ref[...] * z1

    f32 = jnp.float32
    return pl.pallas_call(
        body,
        grid=(N // R,),
        in_specs=[
            pl.BlockSpec((NC, R, H), lambda i: (0, i, 0)),
            pl.BlockSpec((R, H), lambda i: (i, 0)),
            pl.BlockSpec((R, 1), lambda i: (i, 0)),
            pl.BlockSpec((R, 1), lambda i: (i, 0)),
            pl.BlockSpec((1, H), lambda i: (0, 0)),
        ],
        out_specs=(
            pl.BlockSpec((R, H), lambda i: (i, 0)),
            pl.BlockSpec((R, H), lambda i: (i, 0)),
        ),
        out_shape=(
            jax.ShapeDtypeStruct((N, H), f32),
            jax.ShapeDtypeStruct((N, H), f32),
        ),
    )(acc1, h1, dis, dinv, b1)


def _tc3(acc2, z1, dis, dinv, W2, b2, N, N2, H, C, R):
    def body(acc_ref, z1_ref, dis_ref, dinv_ref, w2_ref, b2_ref, out_ref):
        aggz = (dis_ref[...] * (acc_ref[0] + acc_ref[1])
                + dinv_ref[...] * z1_ref[...])
        o = jnp.dot(aggz, w2_ref[...], preferred_element_type=jnp.float32)
        o = o + b2_ref[...]
        m = jnp.max(o, axis=1, keepdims=True)
        lse = m + jnp.log(jnp.sum(jnp.exp(o - m), axis=1, keepdims=True))
        out_ref[...] = o - lse

    return pl.pallas_call(
        body,
        grid=(N // R,),
        in_specs=[
            pl.BlockSpec((NC, R, H), lambda i: (0, i, 0)),
            pl.BlockSpec((R, H), lambda i: (i, 0)),
            pl.BlockSpec((R, 1), lambda i: (i, 0)),
            pl.BlockSpec((R, 1), lambda i: (i, 0)),
            pl.BlockSpec((H, C), lambda i: (0, 0)),
            pl.BlockSpec((1, C), lambda i: (0, 0)),
        ],
        out_specs=pl.BlockSpec((R, C), lambda i: (i, 0)),
        out_shape=jax.ShapeDtypeStruct((N, C), jnp.float32),
    )(acc2, z1, dis, dinv, W2, b2)


def kernel(x, edge_index, edge_weight, W1, b1, W2, b2):
    N, D = x.shape
    H = W1.shape[1]
    C = W2.shape[1]
    E = edge_weight.shape[0]
    NW = NC * NS
    N2 = ((N + NS * 8 - 1) // (NS * 8)) * (NS * 8)

    assert E % (NW * B * NBUF) == 0, "edge count not divisible by chunking"
    EW = E // NW
    K = EW // B

    s1 = edge_index[0]
    d1 = edge_index[1]
    zeros_h = jnp.zeros((N2, H), jnp.float32)

    R = 2000 if N % 2000 == 0 else N
    degp = _make_deg_kernel(N2, H, K, EW)(d1, edge_weight, zeros_h)
    dis, dinv, h1, t1 = _tc1(degp, x, W1, N, N2, D, H, R)
    msg = _make_msg_kernel(N, N2, H, K, EW)
    acc1 = msg(t1, s1, d1, edge_weight, zeros_h)
    z1, t2 = _tc2(acc1, h1, dis, dinv, b1.reshape(1, H), N, N2, H, R)
    acc2 = msg(t2, s1, d1, edge_weight, zeros_h)
    return _tc3(acc2, z1, dis, dinv, W2, b2.reshape(1, C), N, N2, H, C, R)


# speedup vs baseline: 1.7010x; 1.7010x over previous
import functools

import jax
import jax.numpy as jnp
from jax import lax
from jax.experimental import pallas as pl
from jax.experimental.pallas import tpu as pltpu
from jax.experimental.pallas import tpu_sc as plsc

NC = 2
NS = 16
NBUF = 8
LEAD = 5


def _scale_rows(rows_ref, wv, j, Bp):
    nt = Bp // 16
    tail = Bp - nt * 16

    def scale(g2, c2):
        wvec = wv[j, pl.ds(g2 * 16, 16)]
        for i in range(16):
            e = g2 * 16 + i
            rows_ref[e, :] = rows_ref[e, :] * wvec[i]
        return c2

    lax.fori_loop(0, nt, scale, 0)
    if tail:
        wvec = wv[j, pl.ds(Bp - 16, 16)]
        for i in range(16 - tail, 16):
            e = Bp - 16 + i
            rows_ref[e, :] = rows_ref[e, :] * wvec[i]


def _fill_rows(rows_ref, wv, j, Bp):
    nt = Bp // 16
    tail = Bp - nt * 16

    def fill(g2, c2):
        wvec = wv[j, pl.ds(g2 * 16, 16)]
        for i in range(16):
            rows_ref[g2 * 16 + i, :] = jnp.broadcast_to(wvec[i], (16,))
        return c2

    lax.fori_loop(0, nt, fill, 0)
    if tail:
        wvec = wv[j, pl.ds(Bp - 16, 16)]
        for i in range(16 - tail, 16):
            rows_ref[Bp - 16 + i, :] = jnp.broadcast_to(wvec[i], (16,))


def _make_deg_kernel(N2, H, K, Bp):
    rps = N2 // NS
    mesh = plsc.VectorSubcoreMesh(
        core_axis_name="c", subcore_axis_name="s", num_cores=NC, num_subcores=NS)

    @functools.partial(
        pl.kernel,
        out_type=jax.ShapeDtypeStruct((NC, N2, H), jnp.float32),
        mesh=mesh,
        scratch_types=(
            [pltpu.VMEM((K, Bp), jnp.int32),
             pltpu.VMEM((K, Bp), jnp.float32)]
            + [pltpu.VMEM((((Bp + 15) // 16) * 16, H), jnp.float32)] * NBUF
            + [pltpu.VMEM_SHARED((N2, H), jnp.float32)]
            + [pltpu.SemaphoreType.DMA] * NBUF
        ),
        compiler_params=pltpu.CompilerParams(use_tc_tiling_on_sc=False),
    )
    def deg_kernel(ei_h, w_h, z_h, out_h, dv, wv, *rest):
        rows = rest[:NBUF]
        accn = rest[NBUF]
        ssem = rest[NBUF + 1:]
        c = lax.axis_index("c")
        sid = lax.axis_index("s")
        wid = sid * NC + c
        pltpu.sync_copy(z_h.at[pl.ds(sid * rps, rps)],
                        accn.at[pl.ds(sid * rps, rps)])
        plsc.subcore_barrier()
        pltpu.sync_copy(ei_h.at[1, wid], dv)
        pltpu.sync_copy(w_h.at[wid], wv)

        def outer(g, carry):
            for b in range(NBUF):
                j = g * NBUF + b

                @pl.when(j >= NBUF)
                def _wait_prev():
                    pltpu.make_async_copy(
                        rows[b].at[pl.ds(0, Bp)],
                        accn.at[dv.at[j - NBUF]], ssem[b]).wait()

                _fill_rows(rows[b], wv, j, Bp)
                pltpu.async_copy(rows[b].at[pl.ds(0, Bp)],
                                 accn.at[dv.at[j]], ssem[b], add=True)
            return carry

        lax.fori_loop(0, K // NBUF, outer, 0)
        for b in range(NBUF):
            pltpu.make_async_copy(
                rows[b].at[pl.ds(0, Bp)],
                accn.at[dv.at[K - NBUF + b]], ssem[b]).wait()
        plsc.subcore_barrier()
        pltpu.sync_copy(accn.at[pl.ds(sid * rps, rps)],
                        out_h.at[c, pl.ds(sid * rps, rps)])

    return deg_kernel


def _make_msg_kernel(N, N2, H, K, Bp):
    rps = N2 // NS
    mesh = plsc.VectorSubcoreMesh(
        core_axis_name="c", subcore_axis_name="s", num_cores=NC, num_subcores=NS)

    @functools.partial(
        pl.kernel,
        out_type=jax.ShapeDtypeStruct((NC, N2, H), jnp.float32),
        mesh=mesh,
        scratch_types=(
            [pltpu.VMEM((K, Bp), jnp.int32),
             pltpu.VMEM((K, Bp), jnp.int32),
             pltpu.VMEM((K, Bp), jnp.float32)]
            + [pltpu.VMEM((((Bp + 15) // 16) * 16, H), jnp.float32)] * NBUF
            + [pltpu.VMEM_SHARED((N2, H), jnp.float32)]
            + [pltpu.SemaphoreType.DMA] * NBUF
            + [pltpu.SemaphoreType.DMA] * NBUF
        ),
        compiler_params=pltpu.CompilerParams(use_tc_tiling_on_sc=False),
    )
    def msg_kernel(table_h, ei_h, w_h, z_h, out_h, sv, dv, wv, *rest):
        rows = rest[:NBUF]
        acc = rest[NBUF]
        gsem = rest[NBUF + 1:NBUF + 1 + NBUF]
        ssem = rest[NBUF + 1 + NBUF:]
        c = lax.axis_index("c")
        sid = lax.axis_index("s")
        wid = sid * NC + c
        pltpu.sync_copy(z_h.at[pl.ds(sid * rps, rps)],
                        acc.at[pl.ds(sid * rps, rps)])
        plsc.subcore_barrier()
        pltpu.sync_copy(ei_h.at[0, wid], sv)
        pltpu.sync_copy(ei_h.at[1, wid], dv)
        pltpu.sync_copy(w_h.at[wid], wv)

        for b in range(NBUF):
            pltpu.async_copy(table_h.at[sv.at[b]],
                             rows[b].at[pl.ds(0, Bp)], gsem[b])

        def outer(g, carry):
            for b in range(NBUF):
                j = g * NBUF + b
                bn = (b + LEAD) % NBUF

                @pl.when(jnp.logical_and(j + LEAD >= NBUF, j + LEAD < K))
                def _fire_next():
                    pltpu.make_async_copy(
                        rows[bn].at[pl.ds(0, Bp)],
                        acc.at[dv.at[j + LEAD - NBUF]], ssem[bn]).wait()
                    pltpu.async_copy(table_h.at[sv.at[j + LEAD]],
                                     rows[bn].at[pl.ds(0, Bp)], gsem[bn])

                pltpu.make_async_copy(
                    table_h.at[sv.at[j]],
                    rows[b].at[pl.ds(0, Bp)], gsem[b]).wait()
                _scale_rows(rows[b], wv, j, Bp)
                pltpu.async_copy(rows[b].at[pl.ds(0, Bp)],
                                 acc.at[dv.at[j]], ssem[b], add=True)
            return carry

        lax.fori_loop(0, K // NBUF, outer, 0)
        for b in range(NBUF):
            pltpu.make_async_copy(
                rows[b].at[pl.ds(0, Bp)],
                acc.at[dv.at[K - NBUF + b]], ssem[b]).wait()
        plsc.subcore_barrier()
        pltpu.sync_copy(acc.at[pl.ds(sid * rps, rps)],
                        out_h.at[c, pl.ds(sid * rps, rps)])

    return msg_kernel


def _tc1(degp_v, x3, W1B, P, PV, D, H, R):
    nb = 128 // H

    def body(degp_ref, x_ref, w1b_ref, dis_ref, dinv_ref, h1_ref, t1_ref):
        deg = degp_ref[0, :P] + degp_ref[1, :P] + 1.0
        dis = lax.rsqrt(deg)
        dis_ref[...] = dis
        dinv_ref[...] = 1.0 / deg
        h1 = jnp.dot(x_ref[:, 0, :], w1b_ref[0],
                     preferred_element_type=jnp.float32)
        for b in range(1, nb):
            h1 = h1 + jnp.dot(x_ref[:, b, :], w1b_ref[b],
                              preferred_element_type=jnp.float32)
        h1_ref[...] = h1
        t1_ref[...] = dis * h1

    f32 = jnp.float32
    return pl.pallas_call(
        body,
        grid=(P // R,),
        in_specs=[
            pl.BlockSpec((NC, PV, 128), lambda i: (0, i, 0)),
            pl.BlockSpec((R, nb, D), lambda i: (i, 0, 0)),
            pl.BlockSpec((nb, D, 128), lambda i: (0, 0, 0)),
        ],
        out_specs=tuple(pl.BlockSpec((R, 128), lambda i: (i, 0))
                        for _ in range(4)),
        out_shape=tuple(jax.ShapeDtypeStruct((P, 128), f32)
                        for _ in range(4)),
    )(degp_v, x3, W1B)


def _tc2(acc1_v, h1p, disr, dinvr, b1t, P, PV, H, R):
    def body(acc_ref, h1_ref, dis_ref, dinv_ref, b1_ref, z1_ref, t2_ref):
        agg = (dis_ref[...] * (acc_ref[0, :P] + acc_ref[1, :P])
               + dinv_ref[...] * h1_ref[...] + b1_ref[...])
        z1 = jnp.maximum(agg, 0.0)
        z1_ref[...] = z1
        t2_ref[...] = dis_ref[...] * z1

    f32 = jnp.float32
    return pl.pallas_call(
        body,
        grid=(P // R,),
        in_specs=[
            pl.BlockSpec((NC, PV, 128), lambda i: (0, i, 0)),
            pl.BlockSpec((R, 128), lambda i: (i, 0)),
            pl.BlockSpec((R, 128), lambda i: (i, 0)),
            pl.BlockSpec((R, 128), lambda i: (i, 0)),
            pl.BlockSpec((1, 128), lambda i: (0, 0)),
        ],
        out_specs=(
            pl.BlockSpec((R, 128), lambda i: (i, 0)),
            pl.BlockSpec((R, 128), lambda i: (i, 0)),
        ),
        out_shape=(
            jax.ShapeDtypeStruct((P, 128), f32),
            jax.ShapeDtypeStruct((P, 128), f32),
        ),
    )(acc1_v, h1p, disr, dinvr, b1t)


def _tc3(acc2_v, z1p, disr, dinvr, W2B, b2t, P, PV, H, C, R):
    nb = 128 // H
    CW = nb * C

    def body(acc_ref, z1_ref, dis_ref, dinv_ref, w2b_ref, b2_ref, out_ref):
        aggz = (dis_ref[...] * (acc_ref[0, :P] + acc_ref[1, :P])
                + dinv_ref[...] * z1_ref[...])
        o = jnp.dot(aggz, w2b_ref[...], preferred_element_type=jnp.float32)
        o = o + b2_ref[...]
        lane = lax.broadcasted_iota(jnp.int32, (CW, CW), 0)
        col = lax.broadcasted_iota(jnp.int32, (CW, CW), 1)
        grp = col // C
        os = [o]
        for t in range(1, C):
            tgt = grp * C + (col % C + t) % C
            pm = (lane == tgt).astype(jnp.float32)
            os.append(jnp.dot(o, pm, preferred_element_type=jnp.float32))
        m = os[0]
        for t in range(1, C):
            m = jnp.maximum(m, os[t])
        sume = jnp.exp(os[0] - m)
        for t in range(1, C):
            sume = sume + jnp.exp(os[t] - m)
        out_ref[...] = o - (m + jnp.log(sume))

    return pl.pallas_call(
        body,
        grid=(P // R,),
        in_specs=[
            pl.BlockSpec((NC, PV, 128), lambda i: (0, i, 0)),
            pl.BlockSpec((R, 128), lambda i: (i, 0)),
            pl.BlockSpec((R, 128), lambda i: (i, 0)),
            pl.BlockSpec((R, 128), lambda i: (i, 0)),
            pl.BlockSpec((128, CW), lambda i: (0, 0)),
            pl.BlockSpec((1, CW), lambda i: (0, 0)),
        ],
        out_specs=pl.BlockSpec((R, CW), lambda i: (i, 0)),
        out_shape=jax.ShapeDtypeStruct((P, CW), jnp.float32),
    )(acc2_v, z1p, disr, dinvr, W2B, b2t)


def _pick_chunking(E, NW):
    for Bp in range(128, 15, -1):
        if E % (NW * Bp) == 0 and (E // (NW * Bp)) % NBUF == 0:
            return Bp, E // (NW * Bp), 0
    Bp = 128
    K = ((-(-E // (NW * Bp)) + NBUF - 1) // NBUF) * NBUF
    return Bp, K, NW * K * Bp - E


def kernel(x, edge_index, edge_weight, W1, b1, W2, b2):
    N, D = x.shape
    H = W1.shape[1]
    C = W2.shape[1]
    E = edge_weight.shape[0]
    NW = NC * NS
    N2 = ((N + 127) // 128) * 128
    Bp, K, pad_e = _pick_chunking(E, NW)
    nb = 128 // H
    P = N * H // 128
    PV = N2 * H // 128

    if pad_e:
        ei4 = jnp.concatenate(
            [edge_index, jnp.zeros((2, pad_e), jnp.int32)], axis=1
        ).reshape(2, NW, K, Bp)
        w3 = jnp.concatenate(
            [edge_weight, jnp.zeros((pad_e,), jnp.float32)]).reshape(NW, K, Bp)
    else:
        ei4 = edge_index.reshape(2, NW, K, Bp)
        w3 = edge_weight.reshape(NW, K, Bp)
    zeros_h = jnp.zeros((N2, H), jnp.float32)

    x3 = x.reshape(P, nb, D)
    eye = jnp.eye(nb, dtype=jnp.float32)
    W1B = (eye[:, None, :, None] * W1[None, :, None, :]).reshape(nb, D, 128)
    W2B = (eye[:, None, :, None] * W2[None, :, None, :]).reshape(128, nb * C)
    b1t = jnp.tile(b1, nb).reshape(1, 128)
    b2t = jnp.tile(b2, nb).reshape(1, nb * C)

    R = P
    degp = _make_deg_kernel(N2, H, K, Bp)(ei4, w3, zeros_h)
    disr, dinvr, h1p, t1p = _tc1(
        degp.reshape(NC, PV, 128), x3, W1B, P, PV, D, H, R)
    msg = _make_msg_kernel(N, N2, H, K, Bp)
    acc1 = msg(t1p.reshape(N, H), ei4, w3, zeros_h)
    z1p, t2p = _tc2(acc1.reshape(NC, PV, 128), h1p, disr, dinvr,
                    b1t, P, PV, H, R)
    acc2 = msg(t2p.reshape(N, H), ei4, w3, zeros_h)
    outp = _tc3(acc2.reshape(NC, PV, 128), z1p, disr, dinvr,
                W2B, b2t, P, PV, H, C, R)
    return outp.reshape(N, C)


# speedup vs baseline: 1.7518x; 1.0299x over previous
import functools

import jax
import jax.numpy as jnp
from jax import lax
from jax.experimental import pallas as pl
from jax.experimental.pallas import tpu as pltpu
from jax.experimental.pallas import tpu_sc as plsc

NC = 2
NS = 16
NBUF = 8
LEAD = 5


def _scale_rows(rows_ref, wv, j, Bp):
    nt = Bp // 16
    tail = Bp - nt * 16

    def scale(g2, c2):
        wvec = wv[j, pl.ds(g2 * 16, 16)]
        for i in range(16):
            e = g2 * 16 + i
            rows_ref[e, :] = rows_ref[e, :] * wvec[i]
        return c2

    lax.fori_loop(0, nt, scale, 0)
    if tail:
        wvec = wv[j, pl.ds(Bp - 16, 16)]
        for i in range(16 - tail, 16):
            e = Bp - 16 + i
            rows_ref[e, :] = rows_ref[e, :] * wvec[i]


def _fill_rows(rows_ref, wv, j, Bp):
    nt = Bp // 16
    tail = Bp - nt * 16

    def fill(g2, c2):
        wvec = wv[j, pl.ds(g2 * 16, 16)]
        for i in range(16):
            rows_ref[g2 * 16 + i, :] = jnp.broadcast_to(wvec[i], (16,))
        return c2

    lax.fori_loop(0, nt, fill, 0)
    if tail:
        wvec = wv[j, pl.ds(Bp - 16, 16)]
        for i in range(16 - tail, 16):
            rows_ref[Bp - 16 + i, :] = jnp.broadcast_to(wvec[i], (16,))


def _make_deg_kernel(N2, H, K, Bp):
    rps = N2 // NS
    mesh = plsc.VectorSubcoreMesh(
        core_axis_name="c", subcore_axis_name="s", num_cores=NC, num_subcores=NS)

    @functools.partial(
        pl.kernel,
        out_type=jax.ShapeDtypeStruct((NC, N2, H), jnp.float32),
        mesh=mesh,
        scratch_types=(
            [pltpu.VMEM((K, Bp), jnp.int32),
             pltpu.VMEM((K, Bp), jnp.float32)]
            + [pltpu.VMEM((((Bp + 15) // 16) * 16, H), jnp.float32)] * NBUF
            + [pltpu.VMEM_SHARED((N2, H), jnp.float32)]
            + [pltpu.SemaphoreType.DMA] * NBUF
        ),
        compiler_params=pltpu.CompilerParams(use_tc_tiling_on_sc=False),
    )
    def deg_kernel(ei_h, w_h, z_h, out_h, dv, wv, *rest):
        rows = rest[:NBUF]
        accn = rest[NBUF]
        ssem = rest[NBUF + 1:]
        c = lax.axis_index("c")
        sid = lax.axis_index("s")
        wid = sid * NC + c
        pltpu.sync_copy(z_h.at[pl.ds(sid * rps, rps)],
                        accn.at[pl.ds(sid * rps, rps)])
        plsc.subcore_barrier()
        pltpu.sync_copy(ei_h.at[1, wid], dv)
        pltpu.sync_copy(w_h.at[wid], wv)

        def outer(g, carry):
            for b in range(NBUF):
                j = g * NBUF + b

                @pl.when(j >= NBUF)
                def _wait_prev():
                    pltpu.make_async_copy(
                        rows[b].at[pl.ds(0, Bp)],
                        accn.at[dv.at[j - NBUF]], ssem[b]).wait()

                _fill_rows(rows[b], wv, j, Bp)
                pltpu.async_copy(rows[b].at[pl.ds(0, Bp)],
                                 accn.at[dv.at[j]], ssem[b], add=True)
            return carry

        lax.fori_loop(0, K // NBUF, outer, 0)
        for b in range(NBUF):
            pltpu.make_async_copy(
                rows[b].at[pl.ds(0, Bp)],
                accn.at[dv.at[K - NBUF + b]], ssem[b]).wait()
        plsc.subcore_barrier()
        pltpu.sync_copy(accn.at[pl.ds(sid * rps, rps)],
                        out_h.at[c, pl.ds(sid * rps, rps)])

    return deg_kernel


def _make_msg_kernel(N, N2, H, K, Bp):
    rps = N2 // NS
    mesh = plsc.VectorSubcoreMesh(
        core_axis_name="c", subcore_axis_name="s", num_cores=NC, num_subcores=NS)

    @functools.partial(
        pl.kernel,
        out_type=jax.ShapeDtypeStruct((NC, N2, H), jnp.float32),
        mesh=mesh,
        scratch_types=(
            [pltpu.VMEM((K, Bp), jnp.int32),
             pltpu.VMEM((K, Bp), jnp.int32),
             pltpu.VMEM((K, Bp), jnp.float32)]
            + [pltpu.VMEM((((Bp + 15) // 16) * 16, H), jnp.float32)] * NBUF
            + [pltpu.VMEM_SHARED((N2, H), jnp.float32)]
            + [pltpu.SemaphoreType.DMA] * NBUF
            + [pltpu.SemaphoreType.DMA] * NBUF
        ),
        compiler_params=pltpu.CompilerParams(use_tc_tiling_on_sc=False),
    )
    def msg_kernel(table_h, ei_h, w_h, z_h, out_h, sv, dv, wv, *rest):
        rows = rest[:NBUF]
        acc = rest[NBUF]
        gsem = rest[NBUF + 1:NBUF + 1 + NBUF]
        ssem = rest[NBUF + 1 + NBUF:]
        c = lax.axis_index("c")
        sid = lax.axis_index("s")
        wid = sid * NC + c
        pltpu.sync_copy(z_h.at[pl.ds(sid * rps, rps)],
                        acc.at[pl.ds(sid * rps, rps)])
        plsc.subcore_barrier()
        pltpu.sync_copy(ei_h.at[0, wid], sv)
        pltpu.sync_copy(ei_h.at[1, wid], dv)
        pltpu.sync_copy(w_h.at[wid], wv)

        for b in range(NBUF):
            pltpu.async_copy(table_h.at[sv.at[b]],
                             rows[b].at[pl.ds(0, Bp)], gsem[b])

        def outer(g, carry):
            for b in range(NBUF):
                j = g * NBUF + b
                bn = (b + LEAD) % NBUF

                @pl.when(jnp.logical_and(j + LEAD >= NBUF, j + LEAD < K))
                def _fire_next():
                    pltpu.make_async_copy(
                        rows[bn].at[pl.ds(0, Bp)],
                        acc.at[dv.at[j + LEAD - NBUF]], ssem[bn]).wait()
                    pltpu.async_copy(table_h.at[sv.at[j + LEAD]],
                                     rows[bn].at[pl.ds(0, Bp)], gsem[bn])

                pltpu.make_async_copy(
                    table_h.at[sv.at[j]],
                    rows[b].at[pl.ds(0, Bp)], gsem[b]).wait()
                _scale_rows(rows[b], wv, j, Bp)
                pltpu.async_copy(rows[b].at[pl.ds(0, Bp)],
                                 acc.at[dv.at[j]], ssem[b], add=True)
            return carry

        lax.fori_loop(0, K // NBUF, outer, 0)
        for b in range(NBUF):
            pltpu.make_async_copy(
                rows[b].at[pl.ds(0, Bp)],
                acc.at[dv.at[K - NBUF + b]], ssem[b]).wait()
        plsc.subcore_barrier()
        pltpu.sync_copy(acc.at[pl.ds(sid * rps, rps)],
                        out_h.at[c, pl.ds(sid * rps, rps)])

    return msg_kernel


def _tca(x3, W1B, P, D, H):
    nb = 128 // H

    def body(x_ref, w1b_ref, h1_ref):
        h1 = jnp.dot(x_ref[:, 0, :], w1b_ref[0],
                     preferred_element_type=jnp.float32)
        for b in range(1, nb):
            h1 = h1 + jnp.dot(x_ref[:, b, :], w1b_ref[b],
                              preferred_element_type=jnp.float32)
        h1_ref[...] = h1

    return pl.pallas_call(
        body,
        grid=(1,),
        in_specs=[
            pl.BlockSpec((P, nb, D), lambda i: (0, 0, 0)),
            pl.BlockSpec((nb, D, 128), lambda i: (0, 0, 0)),
        ],
        out_specs=pl.BlockSpec((P, 128), lambda i: (0, 0)),
        out_shape=jax.ShapeDtypeStruct((P, 128), jnp.float32),
    )(x3, W1B)


def _tcb(degp_v, h1p, P, PV, H):
    def body(degp_ref, h1_ref, dis_ref, dinv_ref, t1_ref):
        deg = degp_ref[0, :P] + degp_ref[1, :P] + 1.0
        dis = lax.rsqrt(deg)
        dis_ref[...] = dis
        dinv_ref[...] = 1.0 / deg
        t1_ref[...] = dis * h1_ref[...]

    f32 = jnp.float32
    return pl.pallas_call(
        body,
        grid=(1,),
        in_specs=[
            pl.BlockSpec((NC, PV, 128), lambda i: (0, 0, 0)),
            pl.BlockSpec((P, 128), lambda i: (0, 0)),
        ],
        out_specs=tuple(pl.BlockSpec((P, 128), lambda i: (0, 0))
                        for _ in range(3)),
        out_shape=tuple(jax.ShapeDtypeStruct((P, 128), f32)
                        for _ in range(3)),
    )(degp_v, h1p)


def _tc2(acc1_v, h1p, disr, dinvr, b1t, P, PV, H, R):
    def body(acc_ref, h1_ref, dis_ref, dinv_ref, b1_ref, z1_ref, t2_ref):
        agg = (dis_ref[...] * (acc_ref[0, :P] + acc_ref[1, :P])
               + dinv_ref[...] * h1_ref[...] + b1_ref[...])
        z1 = jnp.maximum(agg, 0.0)
        z1_ref[...] = z1
        t2_ref[...] = dis_ref[...] * z1

    f32 = jnp.float32
    return pl.pallas_call(
        body,
        grid=(P // R,),
        in_specs=[
            pl.BlockSpec((NC, PV, 128), lambda i: (0, i, 0)),
            pl.BlockSpec((R, 128), lambda i: (i, 0)),
            pl.BlockSpec((R, 128), lambda i: (i, 0)),
            pl.BlockSpec((R, 128), lambda i: (i, 0)),
            pl.BlockSpec((1, 128), lambda i: (0, 0)),
        ],
        out_specs=(
            pl.BlockSpec((R, 128), lambda i: (i, 0)),
            pl.BlockSpec((R, 128), lambda i: (i, 0)),
        ),
        out_shape=(
            jax.ShapeDtypeStruct((P, 128), f32),
            jax.ShapeDtypeStruct((P, 128), f32),
        ),
    )(acc1_v, h1p, disr, dinvr, b1t)


def _tc3(acc2_v, z1p, disr, dinvr, W2B, b2t, P, PV, H, C, R):
    nb = 128 // H
    CW = nb * C

    def body(acc_ref, z1_ref, dis_ref, dinv_ref, w2b_ref, b2_ref, out_ref):
        aggz = (dis_ref[...] * (acc_ref[0, :P] + acc_ref[1, :P])
                + dinv_ref[...] * z1_ref[...])
        o = jnp.dot(aggz, w2b_ref[...], preferred_element_type=jnp.float32)
        o = o + b2_ref[...]
        lane = lax.broadcasted_iota(jnp.int32, (CW, CW), 0)
        col = lax.broadcasted_iota(jnp.int32, (CW, CW), 1)
        grp = col // C
        os = [o]
        for t in range(1, C):
            tgt = grp * C + (col % C + t) % C
            pm = (lane == tgt).astype(jnp.float32)
            os.append(jnp.dot(o, pm, preferred_element_type=jnp.float32))
        m = os[0]
        for t in range(1, C):
            m = jnp.maximum(m, os[t])
        sume = jnp.exp(os[0] - m)
        for t in range(1, C):
            sume = sume + jnp.exp(os[t] - m)
        out_ref[...] = o - (m + jnp.log(sume))

    return pl.pallas_call(
        body,
        grid=(P // R,),
        in_specs=[
            pl.BlockSpec((NC, PV, 128), lambda i: (0, i, 0)),
            pl.BlockSpec((R, 128), lambda i: (i, 0)),
            pl.BlockSpec((R, 128), lambda i: (i, 0)),
            pl.BlockSpec((R, 128), lambda i: (i, 0)),
            pl.BlockSpec((128, CW), lambda i: (0, 0)),
            pl.BlockSpec((1, CW), lambda i: (0, 0)),
        ],
        out_specs=pl.BlockSpec((R, CW), lambda i: (i, 0)),
        out_shape=jax.ShapeDtypeStruct((P, CW), jnp.float32),
    )(acc2_v, z1p, disr, dinvr, W2B, b2t)


def _pick_chunking(E, NW):
    for Bp in range(128, 15, -1):
        if E % (NW * Bp) == 0 and (E // (NW * Bp)) % NBUF == 0:
            return Bp, E // (NW * Bp), 0
    Bp = 128
    K = ((-(-E // (NW * Bp)) + NBUF - 1) // NBUF) * NBUF
    return Bp, K, NW * K * Bp - E


def kernel(x, edge_index, edge_weight, W1, b1, W2, b2):
    N, D = x.shape
    H = W1.shape[1]
    C = W2.shape[1]
    E = edge_weight.shape[0]
    NW = NC * NS
    N2 = ((N + 127) // 128) * 128
    Bp, K, pad_e = _pick_chunking(E, NW)
    nb = 128 // H
    P = N * H // 128
    PV = N2 * H // 128

    if pad_e:
        ei4 = jnp.concatenate(
            [edge_index, jnp.zeros((2, pad_e), jnp.int32)], axis=1
        ).reshape(2, NW, K, Bp)
        w3 = jnp.concatenate(
            [edge_weight, jnp.zeros((pad_e,), jnp.float32)]).reshape(NW, K, Bp)
    else:
        ei4 = edge_index.reshape(2, NW, K, Bp)
        w3 = edge_weight.reshape(NW, K, Bp)
    zeros_h = jnp.zeros((N2, H), jnp.float32)

    x3 = x.reshape(P, nb, D)
    eye = jnp.eye(nb, dtype=jnp.float32)
    W1B = (eye[:, None, :, None] * W1[None, :, None, :]).reshape(nb, D, 128)
    W2B = (eye[:, None, :, None] * W2[None, :, None, :]).reshape(128, nb * C)
    b1t = jnp.tile(b1, nb).reshape(1, 128)
    b2t = jnp.tile(b2, nb).reshape(1, nb * C)

    R = P
    h1p = _tca(x3, W1B, P, D, H)
    degp = _make_deg_kernel(N2, H, K, Bp)(ei4, w3, zeros_h)
    disr, dinvr, t1p = _tcb(degp.reshape(NC, PV, 128), h1p, P, PV, H)
    msg = _make_msg_kernel(N, N2, H, K, Bp)
    acc1 = msg(t1p.reshape(N, H), ei4, w3, zeros_h)
    z1p, t2p = _tc2(acc1.reshape(NC, PV, 128), h1p, disr, dinvr,
                    b1t, P, PV, H, R)
    acc2 = msg(t2p.reshape(N, H), ei4, w3, zeros_h)
    outp = _tc3(acc2.reshape(NC, PV, 128), z1p, disr, dinvr,
                W2B, b2t, P, PV, H, C, R)
    return outp.reshape(N, C)
